# Initial kernel scaffold; baseline (speedup 1.0000x reference)
import jax
import jax.numpy as jnp
from jax.experimental import pallas as pl


def kernel(x, edge_index, W0):
    raise NotImplementedError("write your pallas kernel here")


# speedup vs baseline: 13.1497x; 13.1497x over previous
import functools

import jax
import jax.numpy as jnp
from jax import lax
from jax.experimental import pallas as pl
from jax.experimental.pallas import tpu as pltpu
from jax.experimental.pallas import tpu_sc as plsc

N = 10000
NP = 10240
E = 320000
NTILES = 32
K = 80
C = 128
QC = 16
QN = K // QC
EP = NTILES * K * C
ROWS_PER_TILE = NP // 16


def _sc_mesh():
    return plsc.VectorSubcoreMesh(core_axis_name="c", subcore_axis_name="s")


def _count_body(col_hbm, zeros_hbm, out_hbm, ones_v, col_v, cnt_sh):
    c = lax.axis_index("c")
    s = lax.axis_index("s")
    wid = s * 2 + c
    for i in range(C // 16):
        ones_v[pl.ds(i * 16, 16)] = jnp.ones((16,), jnp.float32)
    pltpu.sync_copy(col_hbm.at[wid], col_v)
    pltpu.sync_copy(zeros_hbm.at[pl.ds(s * ROWS_PER_TILE, ROWS_PER_TILE)],
                    cnt_sh.at[pl.ds(s * ROWS_PER_TILE, ROWS_PER_TILE)])
    plsc.subcore_barrier()

    def chunk(j, _):
        pltpu.sync_copy(ones_v, cnt_sh.at[col_v.at[j]], add=True)
        return 0

    lax.fori_loop(0, K, chunk, 0)
    plsc.subcore_barrier()
    pltpu.sync_copy(cnt_sh.at[pl.ds(s * ROWS_PER_TILE, ROWS_PER_TILE)],
                    out_hbm.at[c, pl.ds(s * ROWS_PER_TILE, ROWS_PER_TILE)])


def _sc_count(col3, zeros1):
    return pl.kernel(
        _count_body,
        out_type=jax.ShapeDtypeStruct((2, NP), jnp.float32),
        mesh=_sc_mesh(),
        scratch_types=[
            pltpu.VMEM((C,), jnp.float32),
            pltpu.VMEM((K, C), jnp.int32),
            pltpu.VMEM_SHARED((NP,), jnp.float32),
        ],
    )(col3, zeros1)


def _edge_body(row_hbm, col_hbm, z_hbm, zeros_hbm, out_hbm,
               row_v, col_v, rows_a, rows_b, gsem, agg_sh):
    c = lax.axis_index("c")
    s = lax.axis_index("s")
    wid = s * 2 + c
    sl = pl.ds(s * ROWS_PER_TILE, ROWS_PER_TILE)
    pltpu.sync_copy(zeros_hbm.at[sl], agg_sh.at[sl])
    plsc.subcore_barrier()

    def quarter(q, _):
        pltpu.sync_copy(row_hbm.at[wid, pl.ds(q * QC, QC)], row_v)
        pltpu.sync_copy(col_hbm.at[wid, pl.ds(q * QC, QC)], col_v)
        pltpu.make_async_copy(z_hbm.at[row_v.at[0]], rows_a, gsem).start()

        def chunk(j, _):
            def do(cur, nxt):
                @pl.when(j + 1 < QC)
                def _():
                    pltpu.make_async_copy(z_hbm.at[row_v.at[j + 1]], nxt,
                                          gsem).start()
                pltpu.make_async_copy(z_hbm.at[row_v.at[j]], cur, gsem).wait()
                pltpu.sync_copy(cur, agg_sh.at[col_v.at[j]], add=True)

            even = lax.rem(j, 2) == 0

            @pl.when(even)
            def _():
                do(rows_a, rows_b)

            @pl.when(jnp.logical_not(even))
            def _():
                do(rows_b, rows_a)
            return 0

        lax.fori_loop(0, QC, chunk, 0)
        return 0

    lax.fori_loop(0, QN, quarter, 0)
    plsc.subcore_barrier()
    pltpu.sync_copy(agg_sh.at[sl], out_hbm.at[c, sl])


def _sc_edges(row3, col3, z, zeros2):
    return pl.kernel(
        _edge_body,
        out_type=jax.ShapeDtypeStruct((2, NP, 128), jnp.float32),
        mesh=_sc_mesh(),
        scratch_types=[
            pltpu.VMEM((QC, C), jnp.int32),
            pltpu.VMEM((QC, C), jnp.int32),
            pltpu.VMEM((C, 128), jnp.float32),
            pltpu.VMEM((C, 128), jnp.float32),
            pltpu.SemaphoreType.DMA,
            pltpu.VMEM_SHARED((NP, 128), jnp.float32),
        ],
    )(row3, col3, z, zeros2)


def _z_body(x_ref, w_ref, cnt_ref, z_ref):
    deg = cnt_ref[0] + cnt_ref[1] + 1.0
    dinv = lax.rsqrt(deg)
    y = lax.dot_general(x_ref[...], w_ref[...], (((1,), (1,)), ((), ())),
                        preferred_element_type=jnp.float32,
                        precision=lax.Precision.HIGHEST)
    z_ref[...] = y * dinv


def _tc_z(x_pad, w0, cnt3):
    blk = 1024
    return pl.pallas_call(
        _z_body,
        grid=(NP // blk,),
        in_specs=[
            pl.BlockSpec((blk, 128), lambda i: (i, 0)),
            pl.BlockSpec((128, 128), lambda i: (0, 0)),
            pl.BlockSpec((2, blk, 1), lambda i: (0, i, 0)),
        ],
        out_specs=pl.BlockSpec((blk, 128), lambda i: (i, 0)),
        out_shape=jax.ShapeDtypeStruct((NP, 128), jnp.float32),
    )(x_pad, w0, cnt3)


def _out_body(p_ref, z_ref, cnt_ref, o_ref):
    deg = cnt_ref[0] + cnt_ref[1] + 1.0
    dinv = lax.rsqrt(deg)
    agg = p_ref[0] + p_ref[1] + z_ref[...]
    o_ref[...] = jnp.maximum(agg * dinv, 0.0)


def _tc_out(parts, z, cnt3):
    blk = 1000
    return pl.pallas_call(
        _out_body,
        grid=(N // blk,),
        in_specs=[
            pl.BlockSpec((2, blk, 128), lambda i: (0, i, 0)),
            pl.BlockSpec((blk, 128), lambda i: (i, 0)),
            pl.BlockSpec((2, blk, 1), lambda i: (0, i, 0)),
        ],
        out_specs=pl.BlockSpec((blk, 128), lambda i: (i, 0)),
        out_shape=jax.ShapeDtypeStruct((N, 128), jnp.float32),
    )(parts, z, cnt3)


@jax.jit
def kernel(x, edge_index, W0):
    pad = EP - E
    row = jnp.concatenate([edge_index[0], jnp.full((pad,), N, jnp.int32)])
    col = jnp.concatenate([edge_index[1], jnp.full((pad,), N, jnp.int32)])
    row3 = row.reshape(NTILES, K, C)
    col3 = col.reshape(NTILES, K, C)
    x_pad = jnp.pad(x, ((0, NP - N), (0, 0)))

    zeros1 = jnp.zeros((NP,), jnp.float32)
    zeros2 = jnp.zeros((NP, 128), jnp.float32)

    counts = _sc_count(col3, zeros1)
    cnt3 = counts.reshape(2, NP, 1)
    z = _tc_z(x_pad, W0, cnt3)
    parts = _sc_edges(row3, col3, z, zeros2)
    return _tc_out(parts, z, cnt3)
